# Initial kernel scaffold; baseline (speedup 1.0000x reference)
import jax
import jax.numpy as jnp
from jax.experimental import pallas as pl


def kernel(category_ids, table, W, b):
    raise NotImplementedError("write your pallas kernel here")


# speedup vs baseline: 8.2862x; 8.2862x over previous
import functools

import jax
import jax.numpy as jnp
from jax import lax
from jax.experimental import pallas as pl
from jax.experimental.pallas import tpu as pltpu
from jax.experimental.pallas import tpu_sc as plsc

EMBED_DIM = 32
D_MODEL = 128

_IDX_W = 128
_GATHERS_PER_STEP = 8


def _make_sc_gather(n_rows: int, d: int):
  info = plsc.get_sparse_core_info()
  nc, ns = info.num_cores, info.num_subcores
  nw = nc * ns
  rows_per_w = n_rows // nw
  step_rows = _IDX_W * _GATHERS_PER_STEP
  n_steps = rows_per_w // step_rows
  assert rows_per_w % step_rows == 0

  mesh = plsc.VectorSubcoreMesh(core_axis_name="c", subcore_axis_name="s")

  @functools.partial(
      pl.kernel,
      mesh=mesh,
      compiler_params=pltpu.CompilerParams(use_tc_tiling_on_sc=False),
      out_type=jax.ShapeDtypeStruct((n_rows, d), jnp.float32),
      scratch_types=[
          pltpu.VMEM((_GATHERS_PER_STEP, _IDX_W), jnp.int32),
          pltpu.VMEM((step_rows, d), jnp.float32),
          pltpu.SemaphoreType.DMA,
      ],
  )
  def gather_kernel(table_hbm, idx_hbm, out_hbm, idx_v, rows_v, sem):
    wid = lax.axis_index("s") * nc + lax.axis_index("c")
    base_blk = wid * (rows_per_w // _IDX_W)

    def body(j, carry):
      blk0 = base_blk + j * _GATHERS_PER_STEP
      pltpu.sync_copy(idx_hbm.at[pl.ds(blk0, _GATHERS_PER_STEP)], idx_v)
      cps = []
      for r in range(_GATHERS_PER_STEP):
        cps.append(
            pltpu.async_copy(
                table_hbm.at[idx_v.at[r]],
                rows_v.at[pl.ds(r * _IDX_W, _IDX_W)],
                sem,
            ))
      for cp in cps:
        cp.wait()
      pltpu.sync_copy(rows_v, out_hbm.at[pl.ds(blk0 * _IDX_W, step_rows)])
      return carry

    lax.fori_loop(0, n_steps, body, 0)

  return gather_kernel


def _mm_body(x_ref, w_ref, b_ref, o_ref):
  o_ref[...] = (
      jnp.dot(x_ref[...], w_ref[...], preferred_element_type=jnp.float32)
      + b_ref[...])


def _tc_matmul(x, w, b):
  n, k = x.shape
  m = w.shape[1]
  bm = 2048
  assert n % bm == 0
  return pl.pallas_call(
      _mm_body,
      grid=(n // bm,),
      in_specs=[
          pl.BlockSpec((bm, k), lambda i: (i, 0)),
          pl.BlockSpec((k, m), lambda i: (0, 0)),
          pl.BlockSpec((1, m), lambda i: (0, 0)),
      ],
      out_specs=pl.BlockSpec((bm, m), lambda i: (i, 0)),
      out_shape=jax.ShapeDtypeStruct((n, m), jnp.float32),
  )(x, w, b.reshape(1, m))


def kernel(category_ids, table, W, b):
  batch, feats = category_ids.shape
  n_rows = batch * feats
  idx = category_ids.reshape(n_rows // _IDX_W, _IDX_W).astype(jnp.int32)
  gather = _make_sc_gather(n_rows, table.shape[1])
  emb = gather(table, idx)
  out = _tc_matmul(emb, W, b)
  return out.reshape(batch, feats, D_MODEL)


# speedup vs baseline: 9.9399x; 1.1996x over previous
import functools

import jax
import jax.numpy as jnp
from jax import lax
from jax.experimental import pallas as pl
from jax.experimental.pallas import tpu as pltpu
from jax.experimental.pallas import tpu_sc as plsc

EMBED_DIM = 32
D_MODEL = 128
_PACK = 128 // EMBED_DIM

_IDX_W = 128
_GATHERS_PER_STEP = 8


def _make_sc_gather(n_rows: int, d: int):
  info = plsc.get_sparse_core_info()
  nc, ns = info.num_cores, info.num_subcores
  nw = nc * ns
  rows_per_w = n_rows // nw
  step_rows = _IDX_W * _GATHERS_PER_STEP
  n_steps = rows_per_w // step_rows
  w_per_q = nw // _PACK
  assert rows_per_w % step_rows == 0 and nw % _PACK == 0

  mesh = plsc.VectorSubcoreMesh(core_axis_name="c", subcore_axis_name="s")

  @functools.partial(
      pl.kernel,
      mesh=mesh,
      compiler_params=pltpu.CompilerParams(use_tc_tiling_on_sc=False),
      out_type=jax.ShapeDtypeStruct((n_rows // _PACK, d * _PACK), jnp.float32),
      scratch_types=[
          pltpu.VMEM((_GATHERS_PER_STEP, _IDX_W), jnp.int32),
          pltpu.VMEM((step_rows, d), jnp.float32),
          pltpu.SemaphoreType.DMA,
      ],
  )
  def gather_kernel(table_hbm, idx_hbm, out_hbm, idx_v, rows_v, sem):
    wid = lax.axis_index("s") * nc + lax.axis_index("c")
    quarter = wid // w_per_q
    m_base = (wid % w_per_q) * rows_per_w
    base_blk = wid * (rows_per_w // _IDX_W)

    def body(j, carry):
      blk0 = base_blk + j * _GATHERS_PER_STEP
      pltpu.sync_copy(idx_hbm.at[pl.ds(blk0, _GATHERS_PER_STEP)], idx_v)
      cps = []
      for r in range(_GATHERS_PER_STEP):
        cps.append(
            pltpu.async_copy(
                table_hbm.at[idx_v.at[r]],
                rows_v.at[pl.ds(r * _IDX_W, _IDX_W)],
                sem,
            ))
      for cp in cps:
        cp.wait()
      pltpu.sync_copy(
          rows_v,
          out_hbm.at[pl.ds(m_base + j * step_rows, step_rows),
                     pl.ds(quarter * d, d)])
      return carry

    lax.fori_loop(0, n_steps, body, 0)

  return gather_kernel


def _mm_body(x_ref, w_ref, b_ref, o_ref):
  y = (jnp.dot(x_ref[...], w_ref[...], preferred_element_type=jnp.float32)
       + b_ref[...])
  o_ref[...] = y.reshape(o_ref.shape)


def _tc_matmul_packed(xp, w_exp, b, batch, feats):
  m_packed = xp.shape[0]
  kdim = xp.shape[1]
  bb = 32
  rows_pb = bb * feats
  n_i = m_packed // rows_pb
  bq = batch // _PACK
  n_bq = bq // bb
  assert m_packed % rows_pb == 0 and bq % bb == 0 and n_bq == n_i
  return pl.pallas_call(
      _mm_body,
      grid=(n_i, _PACK),
      in_specs=[
          pl.BlockSpec((rows_pb, kdim), lambda i, j: (i, 0)),
          pl.BlockSpec((kdim, D_MODEL), lambda i, j: (0, j)),
          pl.BlockSpec((1, D_MODEL), lambda i, j: (0, 0)),
      ],
      out_specs=pl.BlockSpec(
          (bb, feats, D_MODEL), lambda i, j: (j * n_bq + i, 0, 0)),
      out_shape=jax.ShapeDtypeStruct((batch, feats, D_MODEL), jnp.float32),
  )(xp, w_exp, b.reshape(1, D_MODEL))


def kernel(category_ids, table, W, b):
  batch, feats = category_ids.shape
  n_rows = batch * feats
  idx = category_ids.reshape(n_rows // _IDX_W, _IDX_W).astype(jnp.int32)
  gather = _make_sc_gather(n_rows, table.shape[1])
  xp = gather(table, idx)

  w_exp = jnp.zeros((_PACK * EMBED_DIM, _PACK * D_MODEL), jnp.float32)
  for j in range(_PACK):
    w_exp = w_exp.at[j * EMBED_DIM:(j + 1) * EMBED_DIM,
                     j * D_MODEL:(j + 1) * D_MODEL].set(W)

  return _tc_matmul_packed(xp, w_exp, b, batch, feats)
